# baseline (speedup 1.0000x reference)
import jax
import jax.numpy as jnp
from jax.experimental import pallas as pl

N = 10000
H = 8
NEG_SLOPE = 0.2


def _conv(x, src, dst, num_nodes, W, a_s, a_d, b, heads, out_ch, concat):
    h = (x @ W).reshape(-1, heads, out_ch)
    alpha_s = jnp.sum(h * a_s[None, :, :], axis=-1)
    alpha_d = jnp.sum(h * a_d[None, :, :], axis=-1)
    e = alpha_s[src] + alpha_d[dst]
    e = jnp.where(e > 0, e, NEG_SLOPE * e)
    m = jax.ops.segment_max(e, dst, num_segments=num_nodes)
    m = jnp.where(jnp.isfinite(m), m, 0.0)
    ex = jnp.exp(e - m[dst])
    den = jax.ops.segment_sum(ex, dst, num_segments=num_nodes)
    alpha = ex / (den[dst] + 1e-16)
    msg = h[src] * alpha[:, :, None]
    out = jax.ops.segment_sum(msg, dst, num_segments=num_nodes)
    if concat:
        return out.reshape(num_nodes, heads * out_ch) + b
    return out.mean(axis=1) + b


def kernel(x, edge_index, W1, a_src1, a_dst1, b1, W2, a_src2, a_dst2, b2):
    num_nodes = x.shape[0]
    loops = jnp.arange(num_nodes, dtype=edge_index.dtype)
    ei = jnp.concatenate([edge_index, jnp.stack([loops, loops])], axis=1)
    src, dst = ei[0], ei[1]
    h1 = _conv(x, src, dst, num_nodes, W1, a_src1, a_dst1, b1, H, 128, True)
    h1 = jax.nn.elu(h1)
    h2 = _conv(h1, src, dst, num_nodes, W2, a_src2, a_dst2, b2, H, 128, False)
    return jax.nn.log_softmax(h2, axis=-1)


# speedup vs baseline: 9.5071x; 9.5071x over previous
import functools

import jax
import jax.numpy as jnp
from jax import lax
from jax.experimental import pallas as pl
from jax.experimental.pallas import tpu as pltpu
from jax.experimental.pallas import tpu_sc as plsc

N = 10000
H = 8
C = 128
NEG = 0.2
NCORE = 2
NT = 16
HPC = H // NCORE
CH = 128
RB = 1000
RPT = N // NT

E_TOT = 330000
SB = 8
TCH = 8 * (-(-E_TOT // (NT * CH * 8)))
EP = NT * CH * TCH
KROW = EP // CH


def _tc_layer1(x, W1, a_s, a_d):
    nb = N // RB

    def body(x_ref, w_ref, as_ref, ad_ref, h_ref, ls_ref, ld_ref):
        xb = x_ref[...]
        lss, lds = [], []
        for h in range(H):
            hh = jnp.dot(xb, w_ref[:, h * C:(h + 1) * C],
                         preferred_element_type=jnp.float32)
            h_ref[h] = hh
            lss.append(jnp.dot(hh, as_ref[h])[:, None])
            lds.append(jnp.dot(hh, ad_ref[h])[:, None])
        ls_ref[...] = jnp.concatenate(lss, axis=1)
        ld_ref[...] = jnp.concatenate(lds, axis=1)

    return pl.pallas_call(
        body,
        grid=(nb,),
        in_specs=[pl.BlockSpec((RB, 128), lambda i: (i, 0)),
                  pl.BlockSpec((128, H * C), lambda i: (0, 0)),
                  pl.BlockSpec((H, C), lambda i: (0, 0)),
                  pl.BlockSpec((H, C), lambda i: (0, 0))],
        out_specs=[pl.BlockSpec((H, RB, C), lambda i: (0, i, 0)),
                   pl.BlockSpec((RB, H), lambda i: (i, 0)),
                   pl.BlockSpec((RB, H), lambda i: (i, 0))],
        out_shape=[jax.ShapeDtypeStruct((H, N, C), jnp.float32),
                   jax.ShapeDtypeStruct((N, H), jnp.float32),
                   jax.ShapeDtypeStruct((N, H), jnp.float32)],
    )(x, W1, a_s, a_d)


def _tc_layer2(o1, den1, b1, W2, a_s, a_d):
    nb = N // RB

    def body(o_ref, d_ref, b_ref, w_ref, as_ref, ad_ref,
             h_ref, ls_ref, ld_ref):
        xs = []
        for h in range(H):
            v = o_ref[h] / d_ref[:, h][:, None] + b_ref[h][None, :]
            v = jnp.where(v > 0, v, jnp.exp(jnp.minimum(v, 0.0)) - 1.0)
            xs.append(v)
        lss, lds = [], []
        for hp in range(H):
            acc = jnp.dot(xs[0], w_ref[0:C, hp * C:(hp + 1) * C],
                          preferred_element_type=jnp.float32)
            for h in range(1, H):
                acc = acc + jnp.dot(
                    xs[h], w_ref[h * C:(h + 1) * C, hp * C:(hp + 1) * C],
                    preferred_element_type=jnp.float32)
            h_ref[hp] = acc
            lss.append(jnp.dot(acc, as_ref[hp])[:, None])
            lds.append(jnp.dot(acc, ad_ref[hp])[:, None])
        ls_ref[...] = jnp.concatenate(lss, axis=1)
        ld_ref[...] = jnp.concatenate(lds, axis=1)

    return pl.pallas_call(
        body,
        grid=(nb,),
        in_specs=[pl.BlockSpec((H, RB, C), lambda i: (0, i, 0)),
                  pl.BlockSpec((RB, H), lambda i: (i, 0)),
                  pl.BlockSpec((H, C), lambda i: (0, 0)),
                  pl.BlockSpec((H * C, H * C), lambda i: (0, 0)),
                  pl.BlockSpec((H, C), lambda i: (0, 0)),
                  pl.BlockSpec((H, C), lambda i: (0, 0))],
        out_specs=[pl.BlockSpec((H, RB, C), lambda i: (0, i, 0)),
                   pl.BlockSpec((RB, H), lambda i: (i, 0)),
                   pl.BlockSpec((RB, H), lambda i: (i, 0))],
        out_shape=[jax.ShapeDtypeStruct((H, N, C), jnp.float32),
                   jax.ShapeDtypeStruct((N, H), jnp.float32),
                   jax.ShapeDtypeStruct((N, H), jnp.float32)],
    )(o1, den1, b1, W2, a_s, a_d)


def _tc_final(o2, den2, b2):
    nb = N // RB

    def body(o_ref, d_ref, b_ref, out_ref):
        acc = o_ref[0] / d_ref[:, 0][:, None]
        for h in range(1, H):
            acc = acc + o_ref[h] / d_ref[:, h][:, None]
        v = acc * (1.0 / H) + b_ref[0][None, :]
        m = jnp.max(v, axis=1, keepdims=True)
        lse = jnp.log(jnp.sum(jnp.exp(v - m), axis=1, keepdims=True))
        out_ref[...] = v - m - lse

    return pl.pallas_call(
        body,
        grid=(nb,),
        in_specs=[pl.BlockSpec((H, RB, C), lambda i: (0, i, 0)),
                  pl.BlockSpec((RB, H), lambda i: (i, 0)),
                  pl.BlockSpec((1, C), lambda i: (0, 0))],
        out_specs=pl.BlockSpec((RB, C), lambda i: (i, 0)),
        out_shape=jax.ShapeDtypeStruct((N, C), jnp.float32),
    )(o2, den2, b2)


def _sc_body(feat, lsT, ldT, srcm, dstm, out_hbm, den_hbm,
             acc_sh, den_sh, lsg, ldg, sidx, didx, wv, rows0, rows1,
             zb, denb, gsem0, gsem1, ssem0, ssem1, dsem, lsem):
    c = lax.axis_index("c")
    s = lax.axis_index("s")
    rows = (rows0, rows1)
    gsem = (gsem0, gsem1)
    ssem = (ssem0, ssem1)

    def _zb(i, _):
        zb[pl.ds(i * 16, 16)] = jnp.zeros((16,), jnp.float32)
        return 0
    lax.fori_loop(0, 1024 // 16, _zb, 0)

    for hh in range(HPC):
        head = c * HPC + hh
        headN = (head * N).astype(jnp.int32)

        def _zr(i, _):
            for q in range(C // 16):
                rows0[i, pl.ds(q * 16, 16)] = jnp.zeros((16,), jnp.float32)
            return 0
        lax.fori_loop(0, CH, _zr, 0)

        @pl.when(s < 10)
        def _():
            for k in range(7):
                pltpu.sync_copy(rows0.at[pl.ds(0, CH)],
                                acc_sh.at[pl.ds(s * 1000 + k * CH, CH)])
            pltpu.sync_copy(rows0.at[pl.ds(0, 104)],
                            acc_sh.at[pl.ds(s * 1000 + 7 * CH, 104)])
            pltpu.sync_copy(zb.at[pl.ds(0, 1000)],
                            den_sh.at[pl.ds(s * 1000, 1000)])

        lsT_h = lsT.at[pl.ds(headN, N)]
        ldT_h = ldT.at[pl.ds(headN, N)]
        feat_h = feat.at[pl.ds(headN, N), :]
        plsc.subcore_barrier()

        iota16 = lax.iota(jnp.int32, 16)

        def _gstart(r, buf):
            pltpu.async_copy(feat_h.at[sidx.at[r]], rows[buf], gsem[buf])

        def _gwait(r, buf):
            pltpu.make_async_copy(feat_h.at[sidx.at[r]], rows[buf],
                                  gsem[buf]).wait()

        def _chunk(r, buf, obuf):
            _gwait(r, buf)

            @pl.when(r >= 1)
            def _():
                pltpu.make_async_copy(rows[obuf],
                                      acc_sh.at[didx.at[r - 1]],
                                      ssem[obuf]).wait()
                pltpu.make_async_copy(wv.at[r - 1],
                                      den_sh.at[didx.at[r - 1]],
                                      dsem).wait()

            @pl.when(r + 1 < SB)
            def _():
                _gstart(r + 1, obuf)

            def _scale(j16, _):
                w16 = wv[r, pl.ds(j16 * 16, 16)]
                for i in range(16):
                    w_s = w16[i]
                    for q in range(C // 16):
                        sl = (j16 * 16 + i, pl.ds(q * 16, 16))
                        rows[buf][sl] = rows[buf][sl] * w_s
                return 0
            lax.fori_loop(0, CH // 16, _scale, 0)
            pltpu.async_copy(rows[buf], acc_sh.at[didx.at[r]],
                             ssem[buf], add=True)
            pltpu.async_copy(wv.at[r], den_sh.at[didx.at[r]],
                             dsem, add=True)

        def _super(b, _):
            row0 = s * TCH + b * SB
            pltpu.sync_copy(srcm.at[pl.ds(row0, SB)], sidx)
            pltpu.sync_copy(dstm.at[pl.ds(row0, SB)], didx)

            def _lg(r, _):
                pltpu.async_copy(lsT_h.at[sidx.at[r]], lsg.at[r], lsem)
                pltpu.async_copy(ldT_h.at[didx.at[r]], ldg.at[r], lsem)
                return 0
            lax.fori_loop(0, SB, _lg, 0)

            def _lw(r, _):
                pltpu.make_async_copy(lsT_h.at[sidx.at[r]], lsg.at[r],
                                      lsem).wait()
                pltpu.make_async_copy(ldT_h.at[didx.at[r]], ldg.at[r],
                                      lsem).wait()
                return 0
            lax.fori_loop(0, SB, _lw, 0)

            def _wr(r, _):
                for j in range(CH // 16):
                    sl = (r, pl.ds(j * 16, 16))
                    t = lsg[sl] + ldg[sl]
                    t = jnp.where(t > 0, t, NEG * t)
                    w16 = jnp.exp(t - ldg[sl])
                    eid = (row0 + r) * CH + j * 16 + iota16
                    wv[sl] = jnp.where(eid < E_TOT, w16, 0.0)
                return 0
            lax.fori_loop(0, SB, _wr, 0)

            _gstart(0, 0)

            def _pair(r2, _):
                r = r2 * 2
                _chunk(r, 0, 1)
                _chunk(r + 1, 1, 0)
                return 0
            lax.fori_loop(0, SB // 2, _pair, 0)

            pltpu.make_async_copy(rows[1], acc_sh.at[didx.at[SB - 1]],
                                  ssem[1]).wait()
            pltpu.make_async_copy(wv.at[SB - 1], den_sh.at[didx.at[SB - 1]],
                                  dsem).wait()
            return 0

        lax.fori_loop(0, TCH // SB, _super, 0)
        plsc.subcore_barrier()

        @pl.when(s < 10)
        def _():
            pltpu.sync_copy(acc_sh.at[pl.ds(s * 1000, 1000)],
                            out_hbm.at[pl.ds(headN + s * 1000, 1000)])
            pltpu.sync_copy(den_sh.at[pl.ds(s * 1000, 1000)], denb)
            pltpu.sync_copy(denb,
                            den_hbm.at[pl.ds(headN + s * 1000, 1000)])


def _sc_edge_pass(feat2d, lsT, ldT, srcm, dstm):
    mesh = plsc.VectorSubcoreMesh(core_axis_name="c", subcore_axis_name="s",
                                  num_cores=NCORE, num_subcores=NT)
    k = pl.kernel(
        _sc_body,
        out_type=[jax.ShapeDtypeStruct((H * N, C), jnp.float32),
                  jax.ShapeDtypeStruct((H * N,), jnp.float32)],
        mesh=mesh,
        scratch_types=[
            pltpu.VMEM_SHARED((N, C), jnp.float32),
            pltpu.VMEM_SHARED((N,), jnp.float32),
            pltpu.VMEM((SB, CH), jnp.float32),
            pltpu.VMEM((SB, CH), jnp.float32),
            pltpu.VMEM((SB, CH), jnp.int32),
            pltpu.VMEM((SB, CH), jnp.int32),
            pltpu.VMEM((SB, CH), jnp.float32),
            pltpu.VMEM((CH, C), jnp.float32),
            pltpu.VMEM((CH, C), jnp.float32),
            pltpu.VMEM((1024,), jnp.float32),
            pltpu.VMEM((1000,), jnp.float32),
            pltpu.SemaphoreType.DMA,
            pltpu.SemaphoreType.DMA,
            pltpu.SemaphoreType.DMA,
            pltpu.SemaphoreType.DMA,
            pltpu.SemaphoreType.DMA,
            pltpu.SemaphoreType.DMA,
        ],
        compiler_params=pltpu.CompilerParams(needs_layout_passes=False),
    )
    return k(feat2d, lsT, ldT, srcm, dstm)


def kernel(x, edge_index, W1, a_src1, a_dst1, b1, W2, a_src2, a_dst2, b2):
    loops = jnp.arange(N, dtype=jnp.int32)
    src = jnp.concatenate([edge_index[0], loops])
    dst = jnp.concatenate([edge_index[1], loops])
    pad = EP - E_TOT
    zpad = jnp.zeros((pad,), jnp.int32)
    srcm = jnp.concatenate([src, zpad]).reshape(KROW, CH)
    dstm = jnp.concatenate([dst, zpad]).reshape(KROW, CH)

    hT1, ls1, ld1 = _tc_layer1(x, W1, a_src1, a_dst1)
    out1, den1 = _sc_edge_pass(hT1.reshape(H * N, C),
                               ls1.T.reshape(H * N), ld1.T.reshape(H * N),
                               srcm, dstm)
    hT2, ls2, ld2 = _tc_layer2(out1.reshape(H, N, C),
                               den1.reshape(H, N).T,
                               b1.reshape(H, C), W2, a_src2, a_dst2)
    out2, den2 = _sc_edge_pass(hT2.reshape(H * N, C),
                               ls2.T.reshape(H * N), ld2.T.reshape(H * N),
                               srcm, dstm)
    return _tc_final(out2.reshape(H, N, C), den2.reshape(H, N).T,
                     b2.reshape(1, C))


# speedup vs baseline: 9.7367x; 1.0241x over previous
import functools

import jax
import jax.numpy as jnp
from jax import lax
from jax.experimental import pallas as pl
from jax.experimental.pallas import tpu as pltpu
from jax.experimental.pallas import tpu_sc as plsc

N = 10000
H = 8
C = 128
NEG = 0.2
NCORE = 2
NT = 16
HPC = H // NCORE
CH = 128
RB = 1000
RPT = N // NT

E_TOT = 330000
SB = 24
TCH = 8 * (-(-E_TOT // (NT * CH * 8)))
EP = NT * CH * TCH
KROW = EP // CH


def _tc_layer1(x, W1, a_s, a_d):
    nb = N // RB

    def body(x_ref, w_ref, as_ref, ad_ref, h_ref, ls_ref, ld_ref):
        xb = x_ref[...]
        lss, lds = [], []
        for h in range(H):
            hh = jnp.dot(xb, w_ref[:, h * C:(h + 1) * C],
                         preferred_element_type=jnp.float32)
            h_ref[h] = hh
            lss.append(jnp.dot(hh, as_ref[h])[:, None])
            lds.append(jnp.dot(hh, ad_ref[h])[:, None])
        ls_ref[...] = jnp.concatenate(lss, axis=1)
        ld_ref[...] = jnp.concatenate(lds, axis=1)

    return pl.pallas_call(
        body,
        grid=(nb,),
        in_specs=[pl.BlockSpec((RB, 128), lambda i: (i, 0)),
                  pl.BlockSpec((128, H * C), lambda i: (0, 0)),
                  pl.BlockSpec((H, C), lambda i: (0, 0)),
                  pl.BlockSpec((H, C), lambda i: (0, 0))],
        out_specs=[pl.BlockSpec((H, RB, C), lambda i: (0, i, 0)),
                   pl.BlockSpec((RB, H), lambda i: (i, 0)),
                   pl.BlockSpec((RB, H), lambda i: (i, 0))],
        out_shape=[jax.ShapeDtypeStruct((H, N, C), jnp.float32),
                   jax.ShapeDtypeStruct((N, H), jnp.float32),
                   jax.ShapeDtypeStruct((N, H), jnp.float32)],
    )(x, W1, a_s, a_d)


def _tc_layer2(o1, den1, b1, W2, a_s, a_d):
    nb = N // RB

    def body(o_ref, d_ref, b_ref, w_ref, as_ref, ad_ref,
             h_ref, ls_ref, ld_ref):
        xs = []
        for h in range(H):
            v = o_ref[h] / d_ref[:, h][:, None] + b_ref[h][None, :]
            v = jnp.where(v > 0, v, jnp.exp(jnp.minimum(v, 0.0)) - 1.0)
            xs.append(v)
        lss, lds = [], []
        for hp in range(H):
            acc = jnp.dot(xs[0], w_ref[0:C, hp * C:(hp + 1) * C],
                          preferred_element_type=jnp.float32)
            for h in range(1, H):
                acc = acc + jnp.dot(
                    xs[h], w_ref[h * C:(h + 1) * C, hp * C:(hp + 1) * C],
                    preferred_element_type=jnp.float32)
            h_ref[hp] = acc
            lss.append(jnp.dot(acc, as_ref[hp])[:, None])
            lds.append(jnp.dot(acc, ad_ref[hp])[:, None])
        ls_ref[...] = jnp.concatenate(lss, axis=1)
        ld_ref[...] = jnp.concatenate(lds, axis=1)

    return pl.pallas_call(
        body,
        grid=(nb,),
        in_specs=[pl.BlockSpec((H, RB, C), lambda i: (0, i, 0)),
                  pl.BlockSpec((RB, H), lambda i: (i, 0)),
                  pl.BlockSpec((H, C), lambda i: (0, 0)),
                  pl.BlockSpec((H * C, H * C), lambda i: (0, 0)),
                  pl.BlockSpec((H, C), lambda i: (0, 0)),
                  pl.BlockSpec((H, C), lambda i: (0, 0))],
        out_specs=[pl.BlockSpec((H, RB, C), lambda i: (0, i, 0)),
                   pl.BlockSpec((RB, H), lambda i: (i, 0)),
                   pl.BlockSpec((RB, H), lambda i: (i, 0))],
        out_shape=[jax.ShapeDtypeStruct((H, N, C), jnp.float32),
                   jax.ShapeDtypeStruct((N, H), jnp.float32),
                   jax.ShapeDtypeStruct((N, H), jnp.float32)],
    )(o1, den1, b1, W2, a_s, a_d)


def _tc_final(o2, den2, b2):
    nb = N // RB

    def body(o_ref, d_ref, b_ref, out_ref):
        acc = o_ref[0] / d_ref[:, 0][:, None]
        for h in range(1, H):
            acc = acc + o_ref[h] / d_ref[:, h][:, None]
        v = acc * (1.0 / H) + b_ref[0][None, :]
        m = jnp.max(v, axis=1, keepdims=True)
        lse = jnp.log(jnp.sum(jnp.exp(v - m), axis=1, keepdims=True))
        out_ref[...] = v - m - lse

    return pl.pallas_call(
        body,
        grid=(nb,),
        in_specs=[pl.BlockSpec((H, RB, C), lambda i: (0, i, 0)),
                  pl.BlockSpec((RB, H), lambda i: (i, 0)),
                  pl.BlockSpec((1, C), lambda i: (0, 0))],
        out_specs=pl.BlockSpec((RB, C), lambda i: (i, 0)),
        out_shape=jax.ShapeDtypeStruct((N, C), jnp.float32),
    )(o2, den2, b2)


def _sc_body(feat, lsT, ldT, srcm, dstm, out_hbm, den_hbm,
             acc_sh, den_sh, lsg, ldg, sidx, didx, wv, rows0, rows1,
             zb, denb, gsem0, gsem1, ssem0, ssem1, dsem, lsem):
    c = lax.axis_index("c")
    s = lax.axis_index("s")
    rows = (rows0, rows1)
    gsem = (gsem0, gsem1)
    ssem = (ssem0, ssem1)

    def _zb(i, _):
        zb[pl.ds(i * 16, 16)] = jnp.zeros((16,), jnp.float32)
        return 0
    lax.fori_loop(0, 1024 // 16, _zb, 0)

    for hh in range(HPC):
        head = c * HPC + hh
        headN = (head * N).astype(jnp.int32)

        def _zr(i, _):
            for q in range(C // 16):
                rows0[i, pl.ds(q * 16, 16)] = jnp.zeros((16,), jnp.float32)
            return 0
        lax.fori_loop(0, CH, _zr, 0)

        @pl.when(s < 10)
        def _():
            for k in range(7):
                pltpu.sync_copy(rows0.at[pl.ds(0, CH)],
                                acc_sh.at[pl.ds(s * 1000 + k * CH, CH)])
            pltpu.sync_copy(rows0.at[pl.ds(0, 104)],
                            acc_sh.at[pl.ds(s * 1000 + 7 * CH, 104)])
            pltpu.sync_copy(zb.at[pl.ds(0, 1000)],
                            den_sh.at[pl.ds(s * 1000, 1000)])

        lsT_h = lsT.at[pl.ds(headN, N)]
        ldT_h = ldT.at[pl.ds(headN, N)]
        feat_h = feat.at[pl.ds(headN, N), :]
        plsc.subcore_barrier()

        iota16 = lax.iota(jnp.int32, 16)

        def _gstart(r, buf):
            pltpu.async_copy(feat_h.at[sidx.at[r]], rows[buf], gsem[buf])

        def _gwait(r, buf):
            pltpu.make_async_copy(feat_h.at[sidx.at[r]], rows[buf],
                                  gsem[buf]).wait()

        def _chunk(r, buf, obuf):
            _gwait(r, buf)

            @pl.when(r >= 1)
            def _():
                pltpu.make_async_copy(rows[obuf],
                                      acc_sh.at[didx.at[r - 1]],
                                      ssem[obuf]).wait()
                pltpu.make_async_copy(wv.at[r - 1],
                                      den_sh.at[didx.at[r - 1]],
                                      dsem).wait()

            @pl.when(r + 1 < SB)
            def _():
                _gstart(r + 1, obuf)

            def _scale(j16, _):
                w16 = wv[r, pl.ds(j16 * 16, 16)]
                for i in range(16):
                    w_s = w16[i]
                    for q in range(C // 16):
                        sl = (j16 * 16 + i, pl.ds(q * 16, 16))
                        rows[buf][sl] = rows[buf][sl] * w_s
                return 0
            lax.fori_loop(0, CH // 16, _scale, 0)
            pltpu.async_copy(rows[buf], acc_sh.at[didx.at[r]],
                             ssem[buf], add=True)
            pltpu.async_copy(wv.at[r], den_sh.at[didx.at[r]],
                             dsem, add=True)

        def _super(b, _):
            row0 = s * TCH + b * SB
            pltpu.sync_copy(srcm.at[pl.ds(row0, SB)], sidx)
            pltpu.sync_copy(dstm.at[pl.ds(row0, SB)], didx)

            def _lg(r, _):
                pltpu.async_copy(lsT_h.at[sidx.at[r]], lsg.at[r], lsem)
                pltpu.async_copy(ldT_h.at[didx.at[r]], ldg.at[r], lsem)
                return 0
            lax.fori_loop(0, SB, _lg, 0)

            def _lw(r, _):
                pltpu.make_async_copy(lsT_h.at[sidx.at[r]], lsg.at[r],
                                      lsem).wait()
                pltpu.make_async_copy(ldT_h.at[didx.at[r]], ldg.at[r],
                                      lsem).wait()
                return 0
            lax.fori_loop(0, SB, _lw, 0)

            def _wr(r, _):
                for j in range(CH // 16):
                    sl = (r, pl.ds(j * 16, 16))
                    t = lsg[sl] + ldg[sl]
                    t = jnp.where(t > 0, t, NEG * t)
                    w16 = jnp.exp(t - ldg[sl])
                    eid = (row0 + r) * CH + j * 16 + iota16
                    wv[sl] = jnp.where(eid < E_TOT, w16, 0.0)
                return 0
            lax.fori_loop(0, SB, _wr, 0)

            _gstart(0, 0)

            def _pair(r2, _):
                r = r2 * 2
                _chunk(r, 0, 1)
                _chunk(r + 1, 1, 0)
                return 0
            lax.fori_loop(0, SB // 2, _pair, 0)

            pltpu.make_async_copy(rows[1], acc_sh.at[didx.at[SB - 1]],
                                  ssem[1]).wait()
            pltpu.make_async_copy(wv.at[SB - 1], den_sh.at[didx.at[SB - 1]],
                                  dsem).wait()
            return 0

        lax.fori_loop(0, TCH // SB, _super, 0)
        plsc.subcore_barrier()

        @pl.when(s < 10)
        def _():
            pltpu.sync_copy(acc_sh.at[pl.ds(s * 1000, 1000)],
                            out_hbm.at[pl.ds(headN + s * 1000, 1000)])
            pltpu.sync_copy(den_sh.at[pl.ds(s * 1000, 1000)], denb)
            pltpu.sync_copy(denb,
                            den_hbm.at[pl.ds(headN + s * 1000, 1000)])


def _sc_edge_pass(feat2d, lsT, ldT, srcm, dstm):
    mesh = plsc.VectorSubcoreMesh(core_axis_name="c", subcore_axis_name="s",
                                  num_cores=NCORE, num_subcores=NT)
    k = pl.kernel(
        _sc_body,
        out_type=[jax.ShapeDtypeStruct((H * N, C), jnp.float32),
                  jax.ShapeDtypeStruct((H * N,), jnp.float32)],
        mesh=mesh,
        scratch_types=[
            pltpu.VMEM_SHARED((N, C), jnp.float32),
            pltpu.VMEM_SHARED((N,), jnp.float32),
            pltpu.VMEM((SB, CH), jnp.float32),
            pltpu.VMEM((SB, CH), jnp.float32),
            pltpu.VMEM((SB, CH), jnp.int32),
            pltpu.VMEM((SB, CH), jnp.int32),
            pltpu.VMEM((SB, CH), jnp.float32),
            pltpu.VMEM((CH, C), jnp.float32),
            pltpu.VMEM((CH, C), jnp.float32),
            pltpu.VMEM((1024,), jnp.float32),
            pltpu.VMEM((1000,), jnp.float32),
            pltpu.SemaphoreType.DMA,
            pltpu.SemaphoreType.DMA,
            pltpu.SemaphoreType.DMA,
            pltpu.SemaphoreType.DMA,
            pltpu.SemaphoreType.DMA,
            pltpu.SemaphoreType.DMA,
        ],
        compiler_params=pltpu.CompilerParams(needs_layout_passes=False),
    )
    return k(feat2d, lsT, ldT, srcm, dstm)


def kernel(x, edge_index, W1, a_src1, a_dst1, b1, W2, a_src2, a_dst2, b2):
    loops = jnp.arange(N, dtype=jnp.int32)
    src = jnp.concatenate([edge_index[0], loops])
    dst = jnp.concatenate([edge_index[1], loops])
    pad = EP - E_TOT
    zpad = jnp.zeros((pad,), jnp.int32)
    srcm = jnp.concatenate([src, zpad]).reshape(KROW, CH)
    dstm = jnp.concatenate([dst, zpad]).reshape(KROW, CH)

    hT1, ls1, ld1 = _tc_layer1(x, W1, a_src1, a_dst1)
    out1, den1 = _sc_edge_pass(hT1.reshape(H * N, C),
                               ls1.T.reshape(H * N), ld1.T.reshape(H * N),
                               srcm, dstm)
    hT2, ls2, ld2 = _tc_layer2(out1.reshape(H, N, C),
                               den1.reshape(H, N).T,
                               b1.reshape(H, C), W2, a_src2, a_dst2)
    out2, den2 = _sc_edge_pass(hT2.reshape(H * N, C),
                               ls2.T.reshape(H * N), ld2.T.reshape(H * N),
                               srcm, dstm)
    return _tc_final(out2.reshape(H, N, C), den2.reshape(H, N).T,
                     b2.reshape(1, C))


# speedup vs baseline: 9.9091x; 1.0177x over previous
import functools

import jax
import jax.numpy as jnp
from jax import lax
from jax.experimental import pallas as pl
from jax.experimental.pallas import tpu as pltpu
from jax.experimental.pallas import tpu_sc as plsc

N = 10000
H = 8
C = 128
NEG = 0.2
NCORE = 2
NT = 16
HPC = H // NCORE
CH = 128
RB = 2000
SB = 24

E_TOT = 330000
TCH = 8 * (-(-E_TOT // (NT * CH * 8)))
EP = NT * CH * TCH
KROW = EP // CH


def _tc_layer1(x, W1, a_s, a_d):
    nb = N // RB

    def body(x_ref, w_ref, as_ref, ad_ref, h_ref, ls_ref, ld_ref):
        xb = x_ref[...]
        lss, lds = [], []
        for h in range(H):
            hh = jnp.dot(xb, w_ref[:, h * C:(h + 1) * C],
                         preferred_element_type=jnp.float32)
            h_ref[h] = hh
            lss.append(jnp.dot(hh, as_ref[h])[:, None])
            lds.append(jnp.dot(hh, ad_ref[h])[:, None])
        ls_ref[...] = jnp.concatenate(lss, axis=1)
        ld_ref[...] = jnp.concatenate(lds, axis=1)

    return pl.pallas_call(
        body,
        grid=(nb,),
        in_specs=[pl.BlockSpec((RB, 128), lambda i: (i, 0)),
                  pl.BlockSpec((128, H * C), lambda i: (0, 0)),
                  pl.BlockSpec((H, C), lambda i: (0, 0)),
                  pl.BlockSpec((H, C), lambda i: (0, 0))],
        out_specs=[pl.BlockSpec((H, RB, C), lambda i: (0, i, 0)),
                   pl.BlockSpec((RB, H), lambda i: (i, 0)),
                   pl.BlockSpec((RB, H), lambda i: (i, 0))],
        out_shape=[jax.ShapeDtypeStruct((H, N, C), jnp.float32),
                   jax.ShapeDtypeStruct((N, H), jnp.float32),
                   jax.ShapeDtypeStruct((N, H), jnp.float32)],
    )(x, W1, a_s, a_d)


def _tc_layer2(o1, den1, b1, W2, a_s, a_d):
    nb = N // RB

    def body(o_ref, d_ref, b_ref, w_ref, as_ref, ad_ref,
             h_ref, ls_ref, ld_ref):
        xs = []
        for h in range(H):
            v = o_ref[h] / d_ref[:, h][:, None] + b_ref[h][None, :]
            v = jnp.where(v > 0, v, jnp.exp(jnp.minimum(v, 0.0)) - 1.0)
            xs.append(v)
        lss, lds = [], []
        for hp in range(H):
            acc = jnp.dot(xs[0], w_ref[0:C, hp * C:(hp + 1) * C],
                          preferred_element_type=jnp.float32)
            for h in range(1, H):
                acc = acc + jnp.dot(
                    xs[h], w_ref[h * C:(h + 1) * C, hp * C:(hp + 1) * C],
                    preferred_element_type=jnp.float32)
            h_ref[hp] = acc
            lss.append(jnp.dot(acc, as_ref[hp])[:, None])
            lds.append(jnp.dot(acc, ad_ref[hp])[:, None])
        ls_ref[...] = jnp.concatenate(lss, axis=1)
        ld_ref[...] = jnp.concatenate(lds, axis=1)

    return pl.pallas_call(
        body,
        grid=(nb,),
        in_specs=[pl.BlockSpec((H, RB, C), lambda i: (0, i, 0)),
                  pl.BlockSpec((RB, H), lambda i: (i, 0)),
                  pl.BlockSpec((H, C), lambda i: (0, 0)),
                  pl.BlockSpec((H * C, H * C), lambda i: (0, 0)),
                  pl.BlockSpec((H, C), lambda i: (0, 0)),
                  pl.BlockSpec((H, C), lambda i: (0, 0))],
        out_specs=[pl.BlockSpec((H, RB, C), lambda i: (0, i, 0)),
                   pl.BlockSpec((RB, H), lambda i: (i, 0)),
                   pl.BlockSpec((RB, H), lambda i: (i, 0))],
        out_shape=[jax.ShapeDtypeStruct((H, N, C), jnp.float32),
                   jax.ShapeDtypeStruct((N, H), jnp.float32),
                   jax.ShapeDtypeStruct((N, H), jnp.float32)],
    )(o1, den1, b1, W2, a_s, a_d)


def _tc_final(o2, den2, b2):
    nb = N // RB

    def body(o_ref, d_ref, b_ref, out_ref):
        acc = o_ref[0] / d_ref[:, 0][:, None]
        for h in range(1, H):
            acc = acc + o_ref[h] / d_ref[:, h][:, None]
        v = acc * (1.0 / H) + b_ref[0][None, :]
        m = jnp.max(v, axis=1, keepdims=True)
        lse = jnp.log(jnp.sum(jnp.exp(v - m), axis=1, keepdims=True))
        out_ref[...] = v - m - lse

    return pl.pallas_call(
        body,
        grid=(nb,),
        in_specs=[pl.BlockSpec((H, RB, C), lambda i: (0, i, 0)),
                  pl.BlockSpec((RB, H), lambda i: (i, 0)),
                  pl.BlockSpec((1, C), lambda i: (0, 0))],
        out_specs=pl.BlockSpec((RB, C), lambda i: (i, 0)),
        out_shape=jax.ShapeDtypeStruct((N, C), jnp.float32),
    )(o2, den2, b2)


def _sc_body(feat, lsT, ldT, srcm, dstm, out_hbm, den_hbm,
             acc_sh, den_sh, lsg, ldg, sidx, didx, wv, rb0, rb1,
             zb, denb, gsem0, gsem1, ssem0, ssem1, dsem, lsem):
    c = lax.axis_index("c")
    s = lax.axis_index("s")
    rbb = (rb0, rb1)
    gsem = (gsem0, gsem1)
    ssem = (ssem0, ssem1)

    def _zb(i, _):
        zb[pl.ds(i * 16, 16)] = jnp.zeros((16,), jnp.float32)
        return 0
    lax.fori_loop(0, 1024 // 16, _zb, 0)

    for hh in range(HPC):
        head = c * HPC + hh
        headN = (head * N).astype(jnp.int32)

        def _zr(i, _):
            for q in range(C // 16):
                rb0[i, pl.ds(q * 16, 16)] = jnp.zeros((16,), jnp.float32)
            return 0
        lax.fori_loop(0, CH, _zr, 0)

        @pl.when(s < 10)
        def _():
            for k in range(7):
                pltpu.sync_copy(rb0.at[pl.ds(0, CH)],
                                acc_sh.at[pl.ds(s * 1000 + k * CH, CH)])
            pltpu.sync_copy(rb0.at[pl.ds(0, 104)],
                            acc_sh.at[pl.ds(s * 1000 + 7 * CH, 104)])
            pltpu.sync_copy(zb.at[pl.ds(0, 1000)],
                            den_sh.at[pl.ds(s * 1000, 1000)])

        lsT_h = lsT.at[pl.ds(headN, N)]
        ldT_h = ldT.at[pl.ds(headN, N)]
        feat_h = feat.at[pl.ds(headN, N), :]
        plsc.subcore_barrier()

        iota16 = lax.iota(jnp.int32, 16)

        def _gstart(r, buf):
            pltpu.async_copy(feat_h.at[sidx.at[r]], rbb[buf], gsem[buf])

        def _gwait(r, buf):
            pltpu.make_async_copy(feat_h.at[sidx.at[r]], rbb[buf],
                                  gsem[buf]).wait()

        def _chunk(r, buf, obuf):
            _gwait(r, buf)

            @pl.when(r >= 1)
            def _():
                pltpu.make_async_copy(rbb[obuf],
                                      acc_sh.at[didx.at[r - 1]],
                                      ssem[obuf]).wait()
                pltpu.make_async_copy(wv.at[r - 1],
                                      den_sh.at[didx.at[r - 1]],
                                      dsem).wait()

            @pl.when(r + 1 < SB)
            def _():
                _gstart(r + 1, obuf)

            def _scale(j16, _):
                w16 = wv[r, pl.ds(j16 * 16, 16)]
                for i in range(16):
                    w_s = w16[i]
                    j = j16 * 16 + i
                    for q in range(C // 16):
                        sl = (j, pl.ds(q * 16, 16))
                        rbb[buf][sl] = rbb[buf][sl] * w_s
                return 0
            lax.fori_loop(0, CH // 16, _scale, 0)
            pltpu.async_copy(rbb[buf], acc_sh.at[didx.at[r]],
                             ssem[buf], add=True)
            pltpu.async_copy(wv.at[r], den_sh.at[didx.at[r]],
                             dsem, add=True)

        def _super(b, _):
            row0 = s * TCH + b * SB
            pltpu.sync_copy(srcm.at[pl.ds(row0, SB)], sidx)
            pltpu.sync_copy(dstm.at[pl.ds(row0, SB)], didx)

            def _lg(r, _):
                pltpu.async_copy(lsT_h.at[sidx.at[r]], lsg.at[r], lsem)
                pltpu.async_copy(ldT_h.at[didx.at[r]], ldg.at[r], lsem)
                return 0
            lax.fori_loop(0, SB, _lg, 0)

            def _lw(r, _):
                pltpu.make_async_copy(lsT_h.at[sidx.at[r]], lsg.at[r],
                                      lsem).wait()
                pltpu.make_async_copy(ldT_h.at[didx.at[r]], ldg.at[r],
                                      lsem).wait()
                return 0
            lax.fori_loop(0, SB, _lw, 0)

            _gstart(0, 0)

            def _wr(r, _):
                for j in range(CH // 16):
                    sl = (r, pl.ds(j * 16, 16))
                    t = lsg[sl] + ldg[sl]
                    t = jnp.where(t > 0, t, NEG * t)
                    w16 = jnp.exp(t - ldg[sl])
                    eid = (row0 + r) * CH + j * 16 + iota16
                    wv[sl] = jnp.where(eid < E_TOT, w16, 0.0)
                return 0
            lax.fori_loop(0, SB, _wr, 0)

            def _pair(r2, _):
                r = r2 * 2
                _chunk(r, 0, 1)
                _chunk(r + 1, 1, 0)
                return 0
            lax.fori_loop(0, SB // 2, _pair, 0)

            pltpu.make_async_copy(rbb[1], acc_sh.at[didx.at[SB - 1]],
                                  ssem[1]).wait()
            pltpu.make_async_copy(wv.at[SB - 1], den_sh.at[didx.at[SB - 1]],
                                  dsem).wait()
            return 0

        lax.fori_loop(0, TCH // SB, _super, 0)
        plsc.subcore_barrier()

        @pl.when(s < 10)
        def _():
            pltpu.sync_copy(acc_sh.at[pl.ds(s * 1000, 1000)],
                            out_hbm.at[pl.ds(headN + s * 1000, 1000)])
            pltpu.sync_copy(den_sh.at[pl.ds(s * 1000, 1000)], denb)
            pltpu.sync_copy(denb,
                            den_hbm.at[pl.ds(headN + s * 1000, 1000)])


def _sc_edge_pass(feat2d, lsT, ldT, srcm, dstm):
    mesh = plsc.VectorSubcoreMesh(core_axis_name="c", subcore_axis_name="s",
                                  num_cores=NCORE, num_subcores=NT)
    k = pl.kernel(
        _sc_body,
        out_type=[jax.ShapeDtypeStruct((H * N, C), jnp.float32),
                  jax.ShapeDtypeStruct((H * N,), jnp.float32)],
        mesh=mesh,
        scratch_types=[
            pltpu.VMEM_SHARED((N, C), jnp.float32),
            pltpu.VMEM_SHARED((N,), jnp.float32),
            pltpu.VMEM((SB, CH), jnp.float32),
            pltpu.VMEM((SB, CH), jnp.float32),
            pltpu.VMEM((SB, CH), jnp.int32),
            pltpu.VMEM((SB, CH), jnp.int32),
            pltpu.VMEM((SB, CH), jnp.float32),
            pltpu.VMEM((CH, C), jnp.float32),
            pltpu.VMEM((CH, C), jnp.float32),
            pltpu.VMEM((1024,), jnp.float32),
            pltpu.VMEM((1000,), jnp.float32),
            pltpu.SemaphoreType.DMA,
            pltpu.SemaphoreType.DMA,
            pltpu.SemaphoreType.DMA,
            pltpu.SemaphoreType.DMA,
            pltpu.SemaphoreType.DMA,
            pltpu.SemaphoreType.DMA,
        ],
        compiler_params=pltpu.CompilerParams(needs_layout_passes=False),
    )
    return k(feat2d, lsT, ldT, srcm, dstm)


def kernel(x, edge_index, W1, a_src1, a_dst1, b1, W2, a_src2, a_dst2, b2):
    loops = jnp.arange(N, dtype=jnp.int32)
    src = jnp.concatenate([edge_index[0], loops])
    dst = jnp.concatenate([edge_index[1], loops])
    pad = EP - E_TOT
    zpad = jnp.zeros((pad,), jnp.int32)
    srcm = jnp.concatenate([src, zpad]).reshape(KROW, CH)
    dstm = jnp.concatenate([dst, zpad]).reshape(KROW, CH)

    hT1, ls1, ld1 = _tc_layer1(x, W1, a_src1, a_dst1)
    out1, den1 = _sc_edge_pass(hT1.reshape(H * N, C),
                               ls1.T.reshape(H * N), ld1.T.reshape(H * N),
                               srcm, dstm)
    hT2, ls2, ld2 = _tc_layer2(out1.reshape(H, N, C),
                               den1.reshape(H, N).T,
                               b1.reshape(H, C), W2, a_src2, a_dst2)
    out2, den2 = _sc_edge_pass(hT2.reshape(H * N, C),
                               ls2.T.reshape(H * N), ld2.T.reshape(H * N),
                               srcm, dstm)
    return _tc_final(out2.reshape(H, N, C), den2.reshape(H, N).T,
                     b2.reshape(1, C))


# speedup vs baseline: 9.9126x; 1.0004x over previous
import jax
import jax.numpy as jnp
from jax import lax
from jax.experimental import pallas as pl
from jax.experimental.pallas import tpu as pltpu
from jax.experimental.pallas import tpu_sc as plsc

N = 10000
H = 8
C = 128
NEG = 0.2
NCORE = 2
NT = 16
HPC = H // NCORE
CH = 128
RB = 2000
SB = 24

E_TOT = 330000
TCH = 8 * (-(-E_TOT // (NT * CH * 8)))
EP = NT * CH * TCH
KROW = EP // CH


def _tc_layer1(x, W1, a_s, a_d):
    nb = N // RB

    def body(x_ref, w_ref, as_ref, ad_ref, h_ref, ls_ref, ld_ref):
        xb = x_ref[...]
        lss, lds = [], []
        for h in range(H):
            hh = jnp.dot(xb, w_ref[:, h * C:(h + 1) * C],
                         preferred_element_type=jnp.float32)
            h_ref[h] = hh
            lss.append(jnp.dot(hh, as_ref[h])[:, None])
            lds.append(jnp.dot(hh, ad_ref[h])[:, None])
        ls_ref[...] = jnp.concatenate(lss, axis=1)
        ld_ref[...] = jnp.concatenate(lds, axis=1)

    return pl.pallas_call(
        body,
        grid=(nb,),
        in_specs=[pl.BlockSpec((RB, 128), lambda i: (i, 0)),
                  pl.BlockSpec((128, H * C), lambda i: (0, 0)),
                  pl.BlockSpec((H, C), lambda i: (0, 0)),
                  pl.BlockSpec((H, C), lambda i: (0, 0))],
        out_specs=[pl.BlockSpec((H, RB, C), lambda i: (0, i, 0)),
                   pl.BlockSpec((RB, H), lambda i: (i, 0)),
                   pl.BlockSpec((RB, H), lambda i: (i, 0))],
        out_shape=[jax.ShapeDtypeStruct((H, N, C), jnp.float32),
                   jax.ShapeDtypeStruct((N, H), jnp.float32),
                   jax.ShapeDtypeStruct((N, H), jnp.float32)],
    )(x, W1, a_s, a_d)


def _tc_layer2(o1, den1, b1, W2, a_s, a_d):
    nb = N // RB

    def body(o_ref, d_ref, b_ref, w_ref, as_ref, ad_ref,
             h_ref, ls_ref, ld_ref):
        xs = []
        for h in range(H):
            v = o_ref[h] / d_ref[:, h][:, None] + b_ref[h][None, :]
            v = jnp.where(v > 0, v, jnp.exp(jnp.minimum(v, 0.0)) - 1.0)
            xs.append(v)
        lss, lds = [], []
        for hp in range(H):
            acc = jnp.dot(xs[0], w_ref[0:C, hp * C:(hp + 1) * C],
                          preferred_element_type=jnp.float32)
            for h in range(1, H):
                acc = acc + jnp.dot(
                    xs[h], w_ref[h * C:(h + 1) * C, hp * C:(hp + 1) * C],
                    preferred_element_type=jnp.float32)
            h_ref[hp] = acc
            lss.append(jnp.dot(acc, as_ref[hp])[:, None])
            lds.append(jnp.dot(acc, ad_ref[hp])[:, None])
        ls_ref[...] = jnp.concatenate(lss, axis=1)
        ld_ref[...] = jnp.concatenate(lds, axis=1)

    return pl.pallas_call(
        body,
        grid=(nb,),
        in_specs=[pl.BlockSpec((H, RB, C), lambda i: (0, i, 0)),
                  pl.BlockSpec((RB, H), lambda i: (i, 0)),
                  pl.BlockSpec((H, C), lambda i: (0, 0)),
                  pl.BlockSpec((H * C, H * C), lambda i: (0, 0)),
                  pl.BlockSpec((H, C), lambda i: (0, 0)),
                  pl.BlockSpec((H, C), lambda i: (0, 0))],
        out_specs=[pl.BlockSpec((H, RB, C), lambda i: (0, i, 0)),
                   pl.BlockSpec((RB, H), lambda i: (i, 0)),
                   pl.BlockSpec((RB, H), lambda i: (i, 0))],
        out_shape=[jax.ShapeDtypeStruct((H, N, C), jnp.float32),
                   jax.ShapeDtypeStruct((N, H), jnp.float32),
                   jax.ShapeDtypeStruct((N, H), jnp.float32)],
    )(o1, den1, b1, W2, a_s, a_d)


def _tc_final(o2, den2, b2):
    nb = N // RB

    def body(o_ref, d_ref, b_ref, out_ref):
        acc = o_ref[0] / d_ref[:, 0][:, None]
        for h in range(1, H):
            acc = acc + o_ref[h] / d_ref[:, h][:, None]
        v = acc * (1.0 / H) + b_ref[0][None, :]
        m = jnp.max(v, axis=1, keepdims=True)
        lse = jnp.log(jnp.sum(jnp.exp(v - m), axis=1, keepdims=True))
        out_ref[...] = v - m - lse

    return pl.pallas_call(
        body,
        grid=(nb,),
        in_specs=[pl.BlockSpec((H, RB, C), lambda i: (0, i, 0)),
                  pl.BlockSpec((RB, H), lambda i: (i, 0)),
                  pl.BlockSpec((1, C), lambda i: (0, 0))],
        out_specs=pl.BlockSpec((RB, C), lambda i: (i, 0)),
        out_shape=jax.ShapeDtypeStruct((N, C), jnp.float32),
    )(o2, den2, b2)


def _sc_body(feat, lsT, ldT, srcm, dstm, out_hbm, den_hbm,
             acc_sh, den_sh, lsg, ldg, sidx, didx, wv, rb0, rb1,
             zb, denb, gsem0, gsem1, ssem0, ssem1, dsem, lsem):
    c = lax.axis_index("c")
    s = lax.axis_index("s")
    rbb = (rb0, rb1)
    gsem = (gsem0, gsem1)
    ssem = (ssem0, ssem1)

    def _zb(i, _):
        zb[pl.ds(i * 16, 16)] = jnp.zeros((16,), jnp.float32)
        return 0
    lax.fori_loop(0, 1024 // 16, _zb, 0)

    for hh in range(HPC):
        head = c * HPC + hh
        headN = (head * N).astype(jnp.int32)

        def _zr(i, _):
            for q in range(C // 16):
                rb0[i, pl.ds(q * 16, 16)] = jnp.zeros((16,), jnp.float32)
            return 0
        lax.fori_loop(0, CH, _zr, 0)

        @pl.when(s < 10)
        def _():
            for k in range(7):
                pltpu.sync_copy(rb0.at[pl.ds(0, CH)],
                                acc_sh.at[pl.ds(s * 1000 + k * CH, CH)])
            pltpu.sync_copy(rb0.at[pl.ds(0, 104)],
                            acc_sh.at[pl.ds(s * 1000 + 7 * CH, 104)])
            pltpu.sync_copy(zb.at[pl.ds(0, 1000)],
                            den_sh.at[pl.ds(s * 1000, 1000)])

        lsT_h = lsT.at[pl.ds(headN, N)]
        ldT_h = ldT.at[pl.ds(headN, N)]
        feat_h = feat.at[pl.ds(headN, N), :]
        plsc.subcore_barrier()

        iota16 = lax.iota(jnp.int32, 16)

        def _gstart(r, buf):
            pltpu.async_copy(feat_h.at[sidx.at[r]], rbb[buf], gsem[buf])

        def _gwait(r, buf):
            pltpu.make_async_copy(feat_h.at[sidx.at[r]], rbb[buf],
                                  gsem[buf]).wait()

        def _chunk(r, buf, obuf):
            _gwait(r, buf)

            @pl.when(r >= 1)
            def _():
                pltpu.make_async_copy(rbb[obuf],
                                      acc_sh.at[didx.at[r - 1]],
                                      ssem[obuf]).wait()
                pltpu.make_async_copy(wv.at[r - 1],
                                      den_sh.at[didx.at[r - 1]],
                                      dsem).wait()

            @pl.when(r + 1 < SB)
            def _():
                _gstart(r + 1, obuf)

            def _scale(j16, _):
                w16 = wv[r, pl.ds(j16 * 16, 16)]
                for i in range(16):
                    w_s = w16[i]
                    j = j16 * 16 + i
                    for q in range(C // 16):
                        sl = (j, pl.ds(q * 16, 16))
                        rbb[buf][sl] = rbb[buf][sl] * w_s
                return 0
            lax.fori_loop(0, CH // 16, _scale, 0)
            pltpu.async_copy(rbb[buf], acc_sh.at[didx.at[r]],
                             ssem[buf], add=True)
            pltpu.async_copy(wv.at[r], den_sh.at[didx.at[r]],
                             dsem, add=True)

        def _super(b, _):
            row0 = s * TCH + b * SB
            pltpu.sync_copy(srcm.at[pl.ds(row0, SB)], sidx)
            pltpu.sync_copy(dstm.at[pl.ds(row0, SB)], didx)

            def _lg(r, _):
                pltpu.async_copy(lsT_h.at[sidx.at[r]], lsg.at[r], lsem)
                pltpu.async_copy(ldT_h.at[didx.at[r]], ldg.at[r], lsem)
                return 0
            lax.fori_loop(0, SB, _lg, 0)

            def _lw(r, _):
                pltpu.make_async_copy(lsT_h.at[sidx.at[r]], lsg.at[r],
                                      lsem).wait()
                pltpu.make_async_copy(ldT_h.at[didx.at[r]], ldg.at[r],
                                      lsem).wait()
                return 0
            lax.fori_loop(0, SB, _lw, 0)

            _gstart(0, 0)

            def _wr(r, _):
                for j in range(CH // 16):
                    sl = (r, pl.ds(j * 16, 16))
                    t = lsg[sl] + ldg[sl]
                    t = jnp.where(t > 0, t, NEG * t)
                    w16 = jnp.exp(t - ldg[sl])
                    eid = (row0 + r) * CH + j * 16 + iota16
                    wv[sl] = jnp.where(eid < E_TOT, w16, 0.0)
                return 0
            lax.fori_loop(0, SB, _wr, 0)

            def _pair(r2, _):
                r = r2 * 2
                _chunk(r, 0, 1)
                _chunk(r + 1, 1, 0)
                return 0
            lax.fori_loop(0, SB // 2, _pair, 0)

            pltpu.make_async_copy(rbb[1], acc_sh.at[didx.at[SB - 1]],
                                  ssem[1]).wait()
            pltpu.make_async_copy(wv.at[SB - 1], den_sh.at[didx.at[SB - 1]],
                                  dsem).wait()
            return 0

        lax.fori_loop(0, TCH // SB, _super, 0)
        plsc.subcore_barrier()

        @pl.when(s < 10)
        def _():
            pltpu.sync_copy(acc_sh.at[pl.ds(s * 1000, 1000)],
                            out_hbm.at[pl.ds(headN + s * 1000, 1000)])
            pltpu.sync_copy(den_sh.at[pl.ds(s * 1000, 1000)], denb)
            pltpu.sync_copy(denb,
                            den_hbm.at[pl.ds(headN + s * 1000, 1000)])


def _sc_edge_pass(feat2d, lsT, ldT, srcm, dstm):
    mesh = plsc.VectorSubcoreMesh(core_axis_name="c", subcore_axis_name="s",
                                  num_cores=NCORE, num_subcores=NT)
    k = pl.kernel(
        _sc_body,
        out_type=[jax.ShapeDtypeStruct((H * N, C), jnp.float32),
                  jax.ShapeDtypeStruct((H * N,), jnp.float32)],
        mesh=mesh,
        scratch_types=[
            pltpu.VMEM_SHARED((N, C), jnp.float32),
            pltpu.VMEM_SHARED((N,), jnp.float32),
            pltpu.VMEM((SB, CH), jnp.float32),
            pltpu.VMEM((SB, CH), jnp.float32),
            pltpu.VMEM((SB, CH), jnp.int32),
            pltpu.VMEM((SB, CH), jnp.int32),
            pltpu.VMEM((SB, CH), jnp.float32),
            pltpu.VMEM((CH, C), jnp.float32),
            pltpu.VMEM((CH, C), jnp.float32),
            pltpu.VMEM((1024,), jnp.float32),
            pltpu.VMEM((1000,), jnp.float32),
            pltpu.SemaphoreType.DMA,
            pltpu.SemaphoreType.DMA,
            pltpu.SemaphoreType.DMA,
            pltpu.SemaphoreType.DMA,
            pltpu.SemaphoreType.DMA,
            pltpu.SemaphoreType.DMA,
        ],
        compiler_params=pltpu.CompilerParams(needs_layout_passes=False),
    )
    return k(feat2d, lsT, ldT, srcm, dstm)


def kernel(x, edge_index, W1, a_src1, a_dst1, b1, W2, a_src2, a_dst2, b2):
    loops = jnp.arange(N, dtype=jnp.int32)
    src = jnp.concatenate([edge_index[0], loops])
    dst = jnp.concatenate([edge_index[1], loops])
    pad = EP - E_TOT
    zpad = jnp.zeros((pad,), jnp.int32)
    srcm = jnp.concatenate([src, zpad]).reshape(KROW, CH)
    dstm = jnp.concatenate([dst, zpad]).reshape(KROW, CH)

    hT1, ls1, ld1 = _tc_layer1(x, W1, a_src1, a_dst1)
    out1, den1 = _sc_edge_pass(hT1.reshape(H * N, C),
                               ls1.T.reshape(H * N), ld1.T.reshape(H * N),
                               srcm, dstm)
    hT2, ls2, ld2 = _tc_layer2(out1.reshape(H, N, C),
                               den1.reshape(H, N).T,
                               b1.reshape(H, C), W2, a_src2, a_dst2)
    out2, den2 = _sc_edge_pass(hT2.reshape(H * N, C),
                               ls2.T.reshape(H * N), ld2.T.reshape(H * N),
                               srcm, dstm)
    return _tc_final(out2.reshape(H, N, C), den2.reshape(H, N).T,
                     b2.reshape(1, C))
